# Initial kernel scaffold; baseline (speedup 1.0000x reference)
import jax
import jax.numpy as jnp
from jax.experimental import pallas as pl


def kernel(x, edge_index, hyperedge_attr, W_lin, b_lin, W_sheaf, b_sheaf, W1, b1, W2, b2, W_lin2, b_lin2):
    raise NotImplementedError("write your pallas kernel here")


# speedup vs baseline: 13.9814x; 13.9814x over previous
import functools

import jax
import jax.numpy as jnp
from jax import lax
from jax.experimental import pallas as pl
from jax.experimental.pallas import tpu as pltpu
from jax.experimental.pallas import tpu_sc as plsc

_D = 6
_F = 9
_NC = 2
_NS = 16
_NW = _NC * _NS

def _sc_mesh():
    return plsc.VectorSubcoreMesh(
        core_axis_name="c", subcore_axis_name="s",
        num_cores=_NC, num_subcores=_NS)


def _cdiv(a, b):
    return (a + b - 1) // b


def _mm(A, B, bias, br=512):
    R, K = A.shape
    C = B.shape[1]
    Kp = _cdiv(K, 8) * 8
    if Kp != K:
        A = jnp.pad(A, ((0, 0), (0, Kp - K)))
        B = jnp.pad(B, ((0, Kp - K), (0, 0)))

    def body(a_ref, b_ref, bias_ref, o_ref):
        o_ref[...] = (
            jnp.dot(a_ref[...], b_ref[...], preferred_element_type=jnp.float32)
            + bias_ref[...]
        )

    return pl.pallas_call(
        body,
        grid=(_cdiv(R, br),),
        in_specs=[
            pl.BlockSpec((br, Kp), lambda i: (i, 0)),
            pl.BlockSpec((Kp, C), lambda i: (0, 0)),
            pl.BlockSpec((1, C), lambda i: (0, 0)),
        ],
        out_specs=pl.BlockSpec((br, C), lambda i: (i, 0)),
        out_shape=jax.ShapeDtypeStruct((R, C), jnp.float32),
    )(A, B, bias)


def _lane9_bcast(acc):
    i = lax.broadcasted_iota(jnp.int32, (128, 128), 0)
    j = lax.broadcasted_iota(jnp.int32, (128, 128), 1)
    P = ((j // 16) * 16 + 9 == i).astype(jnp.float32)
    return jnp.dot(acc, P, preferred_element_type=jnp.float32)


def _scale_m(p0, p1):
    S = p0.shape[0]
    G = S * 16 // 128

    def body(r0, r1, o_ref):
        acc = r0[...] + r1[...]
        b = _lane9_bcast(acc)
        binv = jnp.where(b > 0, 1.0 / b, 0.0)
        m = acc * binv
        lane = lax.broadcasted_iota(jnp.int32, m.shape, 1)
        o_ref[...] = jnp.where(lane % 16 == 9, 1.0, m)

    out = pl.pallas_call(
        body,
        out_shape=jax.ShapeDtypeStruct((G, 128), jnp.float32),
    )(p0.reshape(G, 128), p1.reshape(G, 128))
    return out.reshape(S, 16)


def _scale_res(p0, p1, xpT, elu):
    S = p0.shape[0]
    G = S * 16 // 128

    def body(r0, r1, xp_ref, o_ref):
        acc = r0[...] + r1[...]
        dd = _lane9_bcast(acc)
        dinv = jnp.where(dd > 0, 1.0 / dd, 0.0)
        h = acc * dinv + xp_ref[...]
        if elu:
            h = jnp.where(h > 0, h, jnp.exp(h) - 1.0)
        o_ref[...] = h

    out = pl.pallas_call(
        body,
        out_shape=jax.ShapeDtypeStruct((G, 128), jnp.float32),
    )(p0.reshape(G, 128), p1.reshape(G, 128), xpT.reshape(G, 128))
    return out.reshape(S, 16)


def _pad_to_workers(arrs, chunk):
    n = arrs[0].shape[0]
    per = _cdiv(_cdiv(n, _NW), chunk) * chunk
    tot = per * _NW
    return [jnp.pad(a, (0, tot - n)) for a in arrs], per


def _alpha_sc(row, col, srT, scT, CA=512):
    Ep = row.shape[0]
    per = Ep // _NW
    nch = per // CA

    @functools.partial(
        pl.kernel,
        out_type=jax.ShapeDtypeStruct((Ep, 16), jnp.float32),
        mesh=_sc_mesh(),
        compiler_params=pltpu.CompilerParams(use_tc_tiling_on_sc=False),
        scratch_types=[
            pltpu.VMEM((CA,), jnp.int32),
            pltpu.VMEM((CA,), jnp.int32),
            pltpu.VMEM((CA, 16), jnp.float32),
            pltpu.VMEM((CA, 16), jnp.float32),
            pltpu.SemaphoreType.DMA,
            pltpu.SemaphoreType.DMA,
        ],
    )
    def k(row_h, col_h, sr_h, sc_h, out_h, ridx, cidx, av, bv, sem1, sem2):
        c = lax.axis_index("c")
        s = lax.axis_index("s")
        wid = s * _NC + c
        base = wid * per

        def chunk(i, carry):
            off = pl.multiple_of(base + i * CA, 8)
            pltpu.sync_copy(row_h.at[pl.ds(off, CA)], ridx)
            pltpu.sync_copy(col_h.at[pl.ds(off, CA)], cidx)
            cp1 = pltpu.async_copy(sr_h.at[ridx], av, sem1)
            cp2 = pltpu.async_copy(sc_h.at[cidx], bv, sem2)
            cp1.wait()
            cp2.wait()

            def srow(j, cc):
                v = av[j] + bv[j]
                av[j] = 1.0 / (1.0 + jnp.exp(-v))
                return cc

            lax.fori_loop(0, CA, srow, 0)
            pltpu.sync_copy(av, out_h.at[pl.ds(off, CA)])
            return carry

        lax.fori_loop(0, nch, chunk, 0)

    return k(row, col, srT, scT)


def _conv_sc(gidx, sidx, alpha, table, S, CV=512):
    Edp = gidx.shape[0]
    per = Edp // _NW
    nch = per // CV
    rpt = S // _NS

    @functools.partial(
        pl.kernel,
        out_type=jax.ShapeDtypeStruct((2, S, 16), jnp.float32),
        mesh=_sc_mesh(),
        compiler_params=pltpu.CompilerParams(use_tc_tiling_on_sc=False),
        scratch_types=[
            pltpu.VMEM((CV,), jnp.int32),
            pltpu.VMEM((CV,), jnp.int32),
            pltpu.VMEM((CV,), jnp.float32),
            pltpu.VMEM((CV, 16), jnp.float32),
            pltpu.VMEM((rpt, 16), jnp.float32),
            pltpu.VMEM_SHARED((S, 16), jnp.float32),
            pltpu.SemaphoreType.DMA,
        ],
    )
    def k(g_h, s_h, a_h, t_h, out_h, gi, si, alv, rows, dumpb, acc, sem):
        c = lax.axis_index("c")
        s = lax.axis_index("s")
        wid = s * _NC + c

        def z(r, carry):
            dumpb[r] = jnp.zeros((16,), jnp.float32)
            return carry

        lax.fori_loop(0, rpt, z, 0)
        pltpu.sync_copy(dumpb, acc.at[pl.ds(s * rpt, rpt)])
        plsc.subcore_barrier()

        base = wid * per

        def chunk(i, carry):
            off = pl.multiple_of(base + i * CV, 8)
            pltpu.sync_copy(g_h.at[pl.ds(off, CV)], gi)
            pltpu.sync_copy(s_h.at[pl.ds(off, CV)], si)
            pltpu.sync_copy(a_h.at[pl.ds(off, CV)], alv)
            pltpu.async_copy(t_h.at[gi], rows, sem).wait()

            def mul(t, cc):
                a = alv[pl.ds(t * 16, 16)]
                for u in range(16):
                    rows[t * 16 + u] = rows[t * 16 + u] * a[u]
                return cc

            lax.fori_loop(0, CV // 16, mul, 0)
            pltpu.sync_copy(rows, acc.at[si], add=True)
            return carry

        lax.fori_loop(0, nch, chunk, 0)
        plsc.subcore_barrier()
        pltpu.sync_copy(acc.at[pl.ds(s * rpt, rpt)], dumpb)
        pltpu.sync_copy(dumpb, out_h.at[c, pl.ds(s * rpt, rpt)])

    return k(gidx, sidx, alpha, table)


def _pad_table(xp):
    N = xp.shape[0]
    r = xp.reshape(N * _D, _F)
    ones = jnp.ones((N * _D, 1), jnp.float32)
    zeros = jnp.zeros((N * _D, 6), jnp.float32)
    return jnp.concatenate([r, ones, zeros], axis=1)


def kernel(x, edge_index, hyperedge_attr, W_lin, b_lin, W_sheaf, b_sheaf,
           W1, b1, W2, b2, W_lin2, b_lin2):
    d, f = _D, _F
    df = d * f
    N = x.shape[0]
    M = hyperedge_attr.shape[0]
    E = edge_index.shape[1]
    Nd, Md, Ed = N * d, M * d, E * d

    row = edge_index[0].astype(jnp.int32)
    col = edge_index[1].astype(jnp.int32)

    xh = jnp.concatenate([x, hyperedge_attr], axis=0)
    lift = _mm(xh, W_lin, b_lin[None])
    xs, he = lift[:N], lift[N:]

    sr = _mm(xs, W_sheaf[:df], b_sheaf[None])
    sc = _mm(he, W_sheaf[df:], jnp.zeros((1, d), jnp.float32))
    srT = jnp.pad(sr, ((0, 0), (0, 16 - d)))
    scT = jnp.pad(sc, ((0, 0), (0, 16 - d)))

    (row_p, col_p), _ = _pad_to_workers([row, col], 512)
    alpha16 = _alpha_sc(row_p, col_p, srT, scT)
    alpha_flat = alpha16[:E, :d].reshape(Ed)

    offs = jnp.arange(d, dtype=jnp.int32)
    bigrow = (row[:, None] * d + offs[None, :]).reshape(Ed)
    bigcol = (col[:, None] * d + offs[None, :]).reshape(Ed)
    (bigrow_p, bigcol_p, alpha_p), _ = _pad_to_workers(
        [bigrow, bigcol, alpha_flat], 512)

    eye = jnp.eye(d, dtype=jnp.float32)
    W1b = jnp.kron(eye, W1)
    W2b = jnp.kron(eye, W2)
    b1t = jnp.tile(b1, d)[None]
    b2t = jnp.tile(b2, d)[None]

    xp1 = _mm(xs, W1b, b1t)
    xp1T = _pad_table(xp1)
    p = _conv_sc(bigrow_p, bigcol_p, alpha_p, xp1T, Md)
    mT = _scale_m(p[0], p[1])
    q = _conv_sc(bigcol_p, bigrow_p, alpha_p, mT, Nd)
    x1T = _scale_res(q[0], q[1], xp1T, elu=True)
    x1 = x1T[:, :f].reshape(N, df)

    xp2 = _mm(x1, W2b, b2t)
    xp2T = _pad_table(xp2)
    p2 = _conv_sc(bigrow_p, bigcol_p, alpha_p, xp2T, Md)
    m2T = _scale_m(p2[0], p2[1])
    q2 = _conv_sc(bigcol_p, bigrow_p, alpha_p, m2T, Nd)
    x2T = _scale_res(q2[0], q2[1], xp2T, elu=False)
    x2 = x2T[:, :f].reshape(N, df)

    return _mm(x2, W_lin2, b_lin2[None])


# speedup vs baseline: 27.7168x; 1.9824x over previous
import functools

import jax
import jax.numpy as jnp
from jax import lax
from jax.experimental import pallas as pl
from jax.experimental.pallas import tpu as pltpu
from jax.experimental.pallas import tpu_sc as plsc

_D = 6
_F = 9
_NC = 2
_NS = 16
_NW = _NC * _NS


def _sc_mesh():
    return plsc.VectorSubcoreMesh(
        core_axis_name="c", subcore_axis_name="s",
        num_cores=_NC, num_subcores=_NS)


def _cdiv(a, b):
    return (a + b - 1) // b


def _mm(A, B, bias, br=512):
    R, K = A.shape
    C = B.shape[1]
    Kp = _cdiv(K, 8) * 8
    if Kp != K:
        A = jnp.pad(A, ((0, 0), (0, Kp - K)))
        B = jnp.pad(B, ((0, Kp - K), (0, 0)))

    def body(a_ref, b_ref, bias_ref, o_ref):
        o_ref[...] = (
            jnp.dot(a_ref[...], b_ref[...], preferred_element_type=jnp.float32)
            + bias_ref[...]
        )

    return pl.pallas_call(
        body,
        grid=(_cdiv(R, br),),
        in_specs=[
            pl.BlockSpec((br, Kp), lambda i: (i, 0)),
            pl.BlockSpec((Kp, C), lambda i: (0, 0)),
            pl.BlockSpec((1, C), lambda i: (0, 0)),
        ],
        out_specs=pl.BlockSpec((br, C), lambda i: (i, 0)),
        out_shape=jax.ShapeDtypeStruct((R, C), jnp.float32),
    )(A, B, bias)


def _lane9_bcast(acc):
    i = lax.broadcasted_iota(jnp.int32, (128, 128), 0)
    j = lax.broadcasted_iota(jnp.int32, (128, 128), 1)
    P = ((j // 16) * 16 + 9 == i).astype(jnp.float32)
    return jnp.dot(acc, P, preferred_element_type=jnp.float32)


def _scale_m(p0, p1):
    S = p0.shape[0]
    G = S * 96 // 128

    def body(r0, r1, o_ref):
        acc = r0[...] + r1[...]
        b = _lane9_bcast(acc)
        binv = jnp.where(b > 0, 1.0 / b, 0.0)
        m = acc * binv
        lane = lax.broadcasted_iota(jnp.int32, m.shape, 1)
        o_ref[...] = jnp.where(lane % 16 == 9, 1.0, m)

    out = pl.pallas_call(
        body,
        out_shape=jax.ShapeDtypeStruct((G, 128), jnp.float32),
    )(p0.reshape(G, 128), p1.reshape(G, 128))
    return out.reshape(S, 96)


def _scale_res(p0, p1, xpT, elu):
    S = p0.shape[0]
    G = S * 96 // 128

    def body(r0, r1, xp_ref, o_ref):
        acc = r0[...] + r1[...]
        dd = _lane9_bcast(acc)
        dinv = jnp.where(dd > 0, 1.0 / dd, 0.0)
        h = acc * dinv + xp_ref[...]
        if elu:
            h = jnp.where(h > 0, h, jnp.exp(h) - 1.0)
        o_ref[...] = h

    out = pl.pallas_call(
        body,
        out_shape=jax.ShapeDtypeStruct((G, 128), jnp.float32),
    )(p0.reshape(G, 128), p1.reshape(G, 128), xpT.reshape(G, 128))
    return out.reshape(S, 96)


def _pad_to_workers(arrs, chunk):
    n = arrs[0].shape[0]
    per = _cdiv(_cdiv(n, _NW), chunk) * chunk
    tot = per * _NW
    return [jnp.pad(a, (0, tot - n)) for a in arrs], per


def _alpha_sc(row, col, srT, scT, E, CA=512):
    Ep = row.shape[0]
    per = Ep // _NW
    nch = per // CA
    nchH = nch // 2

    @functools.partial(
        pl.kernel,
        out_type=jax.ShapeDtypeStruct((Ep, 16), jnp.float32),
        mesh=_sc_mesh(),
        compiler_params=pltpu.CompilerParams(use_tc_tiling_on_sc=False),
        scratch_types=[
            pltpu.VMEM((CA,), jnp.int32),
            pltpu.VMEM((CA,), jnp.int32),
            pltpu.VMEM((CA,), jnp.int32),
            pltpu.VMEM((CA,), jnp.int32),
            pltpu.VMEM((CA, 16), jnp.float32),
            pltpu.VMEM((CA, 16), jnp.float32),
            pltpu.VMEM((CA, 16), jnp.float32),
            pltpu.VMEM((CA, 16), jnp.float32),
            pltpu.VMEM((CA, 16), jnp.float32),
            pltpu.VMEM((CA, 16), jnp.float32),
            pltpu.SemaphoreType.DMA,
            pltpu.SemaphoreType.DMA,
            pltpu.SemaphoreType.DMA,
            pltpu.SemaphoreType.DMA,
        ],
    )
    def k(row_h, col_h, sr_h, sc_h, out_h,
          ri0, ci0, ri1, ci1, av0, bv0, av1, bv1, ov0, ov1,
          s0a, s0b, s1a, s1b):
        c = lax.axis_index("c")
        s = lax.axis_index("s")
        wid = s * _NC + c
        base = wid * per

        def load_issue(ch, ri, ci, av, bv, sa, sb):
            off = pl.multiple_of(base + ch * CA, 8)
            pltpu.sync_copy(row_h.at[pl.ds(off, CA)], ri)
            pltpu.sync_copy(col_h.at[pl.ds(off, CA)], ci)
            cpa = pltpu.async_copy(sr_h.at[ri], av, sa)
            cpb = pltpu.async_copy(sc_h.at[ci], bv, sb)
            return off, cpa, cpb

        def compute_store(off, cpa, cpb, av, bv, ov):
            cpa.wait()
            cpb.wait()

            def srow(j, cc):
                v = av[j] + bv[j]
                m = jnp.where(off + j < E, 1.0, 0.0)
                ov[j] = m * (1.0 / (1.0 + jnp.exp(-v)))
                return cc

            lax.fori_loop(0, CA, srow, 0)
            pltpu.sync_copy(ov, out_h.at[pl.ds(off, CA)])

        _, cpa0, cpb0 = load_issue(0, ri0, ci0, av0, bv0, s0a, s0b)

        def pairstep(t, carry):
            off0 = pl.multiple_of(base + (2 * t) * CA, 8)
            _, cpa1, cpb1 = load_issue(2 * t + 1, ri1, ci1, av1, bv1, s1a, s1b)
            cpa0 = pltpu.make_async_copy(sr_h.at[ri0], av0, s0a)
            cpb0 = pltpu.make_async_copy(sc_h.at[ci0], bv0, s0b)
            compute_store(off0, cpa0, cpb0, av0, bv0, ov0)
            nxt = lax.rem(2 * t + 2, nch)
            off1 = pl.multiple_of(base + (2 * t + 1) * CA, 8)
            _, _, _ = load_issue(nxt, ri0, ci0, av0, bv0, s0a, s0b)
            compute_store(off1, cpa1, cpb1, av1, bv1, ov1)
            return carry

        lax.fori_loop(0, nchH, pairstep, 0)
        pltpu.make_async_copy(sr_h.at[ri0], av0, s0a).wait()
        pltpu.make_async_copy(sc_h.at[ci0], bv0, s0b).wait()

    return k(row, col, srT, scT)


def _conv_sc(gidx, sidx, alpha, table, S, CV=256):
    Ep = gidx.shape[0]
    per = Ep // _NW
    nch = per // CV
    nchH = nch // 2
    rpt = S // _NS
    DCH = 125
    ndc = rpt // DCH

    @functools.partial(
        pl.kernel,
        out_type=jax.ShapeDtypeStruct((2, S, 96), jnp.float32),
        mesh=_sc_mesh(),
        compiler_params=pltpu.CompilerParams(use_tc_tiling_on_sc=False),
        scratch_types=[
            pltpu.VMEM((CV,), jnp.int32),
            pltpu.VMEM((CV,), jnp.int32),
            pltpu.VMEM((CV,), jnp.int32),
            pltpu.VMEM((CV,), jnp.int32),
            pltpu.VMEM((CV, 16), jnp.float32),
            pltpu.VMEM((CV, 16), jnp.float32),
            pltpu.VMEM((CV, 96), jnp.float32),
            pltpu.VMEM((CV, 96), jnp.float32),
            pltpu.VMEM((DCH, 96), jnp.float32),
            pltpu.VMEM_SHARED((S, 96), jnp.float32),
            pltpu.SemaphoreType.DMA,
            pltpu.SemaphoreType.DMA,
        ],
    )
    def k(g_h, s_h, a_h, t_h, out_h,
          gi0, si0, gi1, si1, al0, al1, r0, r1, dumpb, acc, sem0, sem1):
        c = lax.axis_index("c")
        s = lax.axis_index("s")
        wid = s * _NC + c

        def z(r, carry):
            for kk in range(_D):
                dumpb[r, pl.ds(kk * 16, 16)] = jnp.zeros((16,), jnp.float32)
            return carry

        lax.fori_loop(0, DCH, z, 0)

        def zc(r, carry):
            pltpu.sync_copy(dumpb, acc.at[pl.ds(s * rpt + r * DCH, DCH)])
            return carry

        lax.fori_loop(0, ndc, zc, 0)
        plsc.subcore_barrier()

        base = wid * per

        def load_issue(ch, gi, si, al, rb, sem):
            off = pl.multiple_of(base + ch * CV, 8)
            pltpu.sync_copy(g_h.at[pl.ds(off, CV)], gi)
            pltpu.sync_copy(s_h.at[pl.ds(off, CV)], si)
            pltpu.sync_copy(a_h.at[pl.ds(off, CV)], al)
            pltpu.async_copy(t_h.at[gi], rb, sem)

        def process(gi, si, al, rb, sem):
            pltpu.make_async_copy(t_h.at[gi], rb, sem).wait()

            def mul(i, cc):
                a = al[i]
                for kk in range(_D):
                    sl = pl.ds(kk * 16, 16)
                    rb[i, sl] = rb[i, sl] * a[kk]
                return cc

            lax.fori_loop(0, CV, mul, 0)
            pltpu.sync_copy(rb, acc.at[si], add=True)

        load_issue(0, gi0, si0, al0, r0, sem0)

        def pairstep(t, carry):
            load_issue(2 * t + 1, gi1, si1, al1, r1, sem1)
            process(gi0, si0, al0, r0, sem0)
            nxt = lax.rem(2 * t + 2, nch)
            load_issue(nxt, gi0, si0, al0, r0, sem0)
            process(gi1, si1, al1, r1, sem1)
            return carry

        lax.fori_loop(0, nchH, pairstep, 0)
        pltpu.make_async_copy(t_h.at[gi0], r0, sem0).wait()
        plsc.subcore_barrier()

        def dump(r, carry):
            pltpu.sync_copy(acc.at[pl.ds(s * rpt + r * DCH, DCH)], dumpb)
            pltpu.sync_copy(dumpb, out_h.at[c, pl.ds(s * rpt + r * DCH, DCH)])
            return carry

        lax.fori_loop(0, ndc, dump, 0)

    return k(gidx, sidx, alpha, table)


def _stalk96(W, b, with_one):
    Wp = jnp.pad(W, ((0, 16 - _F), (0, 16 - _F)))
    Wb = jnp.kron(jnp.eye(_D, dtype=jnp.float32), Wp)
    bp = jnp.pad(b, (0, 16 - _F))
    if with_one:
        bp = bp.at[9].set(1.0)
    return Wb, jnp.tile(bp, _D)[None]


def kernel(x, edge_index, hyperedge_attr, W_lin, b_lin, W_sheaf, b_sheaf,
           W1, b1, W2, b2, W_lin2, b_lin2):
    d, f = _D, _F
    df = d * f
    N = x.shape[0]
    M = hyperedge_attr.shape[0]
    E = edge_index.shape[1]

    row = edge_index[0].astype(jnp.int32)
    col = edge_index[1].astype(jnp.int32)

    W_top, W_bot = W_sheaf[:df], W_sheaf[df:]
    z10 = jnp.zeros((x.shape[1], 16 - d), jnp.float32)
    Wn = jnp.concatenate([W_lin, W_lin @ W_top, z10], axis=1)
    bn = jnp.concatenate([b_lin, b_lin @ W_top + b_sheaf,
                          jnp.zeros((16 - d,), jnp.float32)])[None]
    Wh = jnp.concatenate([W_lin, W_lin @ W_bot, z10], axis=1)
    bh = jnp.concatenate([b_lin, b_lin @ W_bot,
                          jnp.zeros((16 - d,), jnp.float32)])[None]
    xs_ext = _mm(x, Wn, bn)
    he_ext = _mm(hyperedge_attr, Wh, bh)
    srT = xs_ext[:, df:]
    scT = he_ext[:, df:]

    (row_p, col_p), _ = _pad_to_workers([row, col], 512)
    alpha16 = _alpha_sc(row_p, col_p, srT, scT, E)

    W1b, b1t = _stalk96(W1, b1, with_one=True)
    W2b, b2t = _stalk96(W2, b2, with_one=True)
    W1x = jnp.zeros((70, 96), jnp.float32)
    W1x = W1x.at[:df].set(
        jnp.kron(jnp.eye(d, dtype=jnp.float32),
                 jnp.pad(W1, ((0, 0), (0, 16 - f)))))
    Wf = jnp.pad(W_lin2.reshape(d, f, -1), ((0, 0), (0, 16 - f), (0, 0))
                 ).reshape(d * 16, -1)

    xp1T = _mm(xs_ext, W1x, b1t)
    p = _conv_sc(row_p, col_p, alpha16, xp1T, M)
    mT = _scale_m(p[0], p[1])
    q = _conv_sc(col_p, row_p, alpha16, mT, N)
    x1T = _scale_res(q[0], q[1], xp1T, elu=True)

    xp2T = _mm(x1T, W2b, b2t)
    p2 = _conv_sc(row_p, col_p, alpha16, xp2T, M)
    m2T = _scale_m(p2[0], p2[1])
    q2 = _conv_sc(col_p, row_p, alpha16, m2T, N)
    x2T = _scale_res(q2[0], q2[1], xp2T, elu=False)

    return _mm(x2T, Wf, b_lin2[None])


# speedup vs baseline: 35.7459x; 1.2897x over previous
import functools

import numpy as np

import jax
import jax.numpy as jnp
from jax import lax
from jax.experimental import pallas as pl
from jax.experimental.pallas import tpu as pltpu
from jax.experimental.pallas import tpu_sc as plsc

_D = 6
_F = 9
_L = 64
_NC = 2
_NS = 16
_NW = _NC * _NS

_PAT = np.array([l // 9 if l < 54 else (l - 54 if l < 60 else 0)
                 for l in range(_L)], dtype=np.int32)


def _sc_mesh():
    return plsc.VectorSubcoreMesh(
        core_axis_name="c", subcore_axis_name="s",
        num_cores=_NC, num_subcores=_NS)


def _cdiv(a, b):
    return (a + b - 1) // b


def _mm(A, B, bias, br=512):
    R, K = A.shape
    C = B.shape[1]
    Kp = _cdiv(K, 8) * 8
    if Kp != K:
        A = jnp.pad(A, ((0, 0), (0, Kp - K)))
        B = jnp.pad(B, ((0, Kp - K), (0, 0)))

    def body(a_ref, b_ref, bias_ref, o_ref):
        o_ref[...] = (
            jnp.dot(a_ref[...], b_ref[...], preferred_element_type=jnp.float32)
            + bias_ref[...]
        )

    return pl.pallas_call(
        body,
        grid=(_cdiv(R, br),),
        in_specs=[
            pl.BlockSpec((br, Kp), lambda i: (i, 0)),
            pl.BlockSpec((Kp, C), lambda i: (0, 0)),
            pl.BlockSpec((1, C), lambda i: (0, 0)),
        ],
        out_specs=pl.BlockSpec((br, C), lambda i: (i, 0)),
        out_shape=jax.ShapeDtypeStruct((R, C), jnp.float32),
    )(A, B, bias)


def _deg_bcast(acc):
    i = lax.broadcasted_iota(jnp.int32, (128, 128), 0)
    j = lax.broadcasted_iota(jnp.int32, (128, 128), 1)
    jm = j % _L
    P = ((jm < 54) & ((j // _L) * _L + 54 + jm // _F == i)).astype(jnp.float32)
    return jnp.dot(acc, P, preferred_element_type=jnp.float32)


def _scale_m(p0, p1):
    S = p0.shape[0]
    G = S * _L // 128

    def body(r0, r1, o_ref):
        acc = r0[...] + r1[...]
        b = _deg_bcast(acc)
        binv = jnp.where(b > 0, 1.0 / b, 0.0)
        m = acc * binv
        lane = lax.broadcasted_iota(jnp.int32, m.shape, 1) % _L
        o_ref[...] = jnp.where((lane >= 54) & (lane < 60), 1.0, m)

    out = pl.pallas_call(
        body,
        out_shape=jax.ShapeDtypeStruct((G, 128), jnp.float32),
    )(p0.reshape(G, 128), p1.reshape(G, 128))
    return out.reshape(S, _L)


def _scale_res(p0, p1, xpT, elu):
    S = p0.shape[0]
    G = S * _L // 128

    def body(r0, r1, xp_ref, o_ref):
        acc = r0[...] + r1[...]
        dd = _deg_bcast(acc)
        dinv = jnp.where(dd > 0, 1.0 / dd, 0.0)
        h = acc * dinv + xp_ref[...]
        if elu:
            h = jnp.where(h > 0, h, jnp.exp(h) - 1.0)
        o_ref[...] = h

    out = pl.pallas_call(
        body,
        out_shape=jax.ShapeDtypeStruct((G, 128), jnp.float32),
    )(p0.reshape(G, 128), p1.reshape(G, 128), xpT.reshape(G, 128))
    return out.reshape(S, _L)


def _pad_to_workers(arrs, chunk):
    n = arrs[0].shape[0]
    per = _cdiv(_cdiv(n, _NW), chunk) * chunk
    tot = per * _NW
    return [jnp.pad(a, (0, tot - n)) for a in arrs], per


def _alpha_sc(row, col, srT, scT, E, CA=512):
    Ep = row.shape[0]
    per = Ep // _NW
    nch = per // CA
    nchH = nch // 2

    @functools.partial(
        pl.kernel,
        out_type=jax.ShapeDtypeStruct((Ep, 16), jnp.float32),
        mesh=_sc_mesh(),
        compiler_params=pltpu.CompilerParams(use_tc_tiling_on_sc=False),
        scratch_types=[
            pltpu.VMEM((CA,), jnp.int32),
            pltpu.VMEM((CA,), jnp.int32),
            pltpu.VMEM((CA,), jnp.int32),
            pltpu.VMEM((CA,), jnp.int32),
            pltpu.VMEM((CA, 16), jnp.float32),
            pltpu.VMEM((CA, 16), jnp.float32),
            pltpu.VMEM((CA, 16), jnp.float32),
            pltpu.VMEM((CA, 16), jnp.float32),
            pltpu.VMEM((CA, 16), jnp.float32),
            pltpu.VMEM((CA, 16), jnp.float32),
            pltpu.SemaphoreType.DMA,
            pltpu.SemaphoreType.DMA,
            pltpu.SemaphoreType.DMA,
            pltpu.SemaphoreType.DMA,
        ],
    )
    def k(row_h, col_h, sr_h, sc_h, out_h,
          ri0, ci0, ri1, ci1, av0, bv0, av1, bv1, ov0, ov1,
          s0a, s0b, s1a, s1b):
        c = lax.axis_index("c")
        s = lax.axis_index("s")
        wid = s * _NC + c
        base = wid * per

        def load_issue(ch, ri, ci, av, bv, sa, sb):
            off = pl.multiple_of(base + ch * CA, 8)
            pltpu.sync_copy(row_h.at[pl.ds(off, CA)], ri)
            pltpu.sync_copy(col_h.at[pl.ds(off, CA)], ci)
            pltpu.async_copy(sr_h.at[ri], av, sa)
            pltpu.async_copy(sc_h.at[ci], bv, sb)

        def compute_store(off, ri, ci, av, bv, ov, sa, sb):
            pltpu.make_async_copy(sr_h.at[ri], av, sa).wait()
            pltpu.make_async_copy(sc_h.at[ci], bv, sb).wait()

            def srow(j, cc):
                v = av[j] + bv[j]
                m = jnp.where(off + j < E, 1.0, 0.0)
                ov[j] = m * (1.0 / (1.0 + jnp.exp(-v)))
                return cc

            lax.fori_loop(0, CA, srow, 0)
            pltpu.sync_copy(ov, out_h.at[pl.ds(off, CA)])

        load_issue(0, ri0, ci0, av0, bv0, s0a, s0b)

        def pairstep(t, carry):
            off0 = pl.multiple_of(base + (2 * t) * CA, 8)
            load_issue(2 * t + 1, ri1, ci1, av1, bv1, s1a, s1b)
            compute_store(off0, ri0, ci0, av0, bv0, ov0, s0a, s0b)
            nxt = lax.rem(2 * t + 2, nch)
            off1 = pl.multiple_of(base + (2 * t + 1) * CA, 8)
            load_issue(nxt, ri0, ci0, av0, bv0, s0a, s0b)
            compute_store(off1, ri1, ci1, av1, bv1, ov1, s1a, s1b)
            return carry

        lax.fori_loop(0, nchH, pairstep, 0)
        pltpu.make_async_copy(sr_h.at[ri0], av0, s0a).wait()
        pltpu.make_async_copy(sc_h.at[ci0], bv0, s0b).wait()

    return k(row, col, srT, scT)


def _conv_sc(gidx, sidx, alpha, table, S, CV=512):
    Ep = gidx.shape[0]
    per = Ep // _NW
    nch = per // CV
    nchH = nch // 2
    rpt = S // _NS
    DCH = 25
    ndc = rpt // DCH

    @functools.partial(
        pl.kernel,
        out_type=jax.ShapeDtypeStruct((2, S, _L), jnp.float32),
        mesh=_sc_mesh(),
        compiler_params=pltpu.CompilerParams(use_tc_tiling_on_sc=False),
        scratch_types=[
            pltpu.VMEM((CV,), jnp.int32),
            pltpu.VMEM((CV,), jnp.int32),
            pltpu.VMEM((CV,), jnp.int32),
            pltpu.VMEM((CV,), jnp.int32),
            pltpu.VMEM((CV, 16), jnp.float32),
            pltpu.VMEM((CV, 16), jnp.float32),
            pltpu.VMEM((CV, _L), jnp.float32),
            pltpu.VMEM((CV, _L), jnp.float32),
            pltpu.VMEM((DCH, _L), jnp.float32),
            pltpu.VMEM((_L // 16, 16), jnp.int32),
            pltpu.VMEM_SHARED((S, _L), jnp.float32),
            pltpu.SemaphoreType.DMA,
            pltpu.SemaphoreType.DMA,
        ],
    )
    def k(g_h, s_h, a_h, t_h, p_h, out_h,
          gi0, si0, gi1, si1, al0, al1, r0, r1, dumpb, patv, acc,
          sem0, sem1):
        c = lax.axis_index("c")
        s = lax.axis_index("s")
        wid = s * _NC + c

        def z(r, carry):
            for kk in range(_L // 16):
                dumpb[r, pl.ds(kk * 16, 16)] = jnp.zeros((16,), jnp.float32)
            return carry

        lax.fori_loop(0, DCH, z, 0)

        def zc(r, carry):
            pltpu.sync_copy(dumpb, acc.at[pl.ds(s * rpt + r * DCH, DCH)])
            return carry

        lax.fori_loop(0, ndc, zc, 0)
        plsc.subcore_barrier()

        pltpu.sync_copy(p_h, patv)
        pats = [patv[j] for j in range(_L // 16)]

        base = wid * per

        def load_issue(ch, gi, si, al, rb, sem):
            off = pl.multiple_of(base + ch * CV, 8)
            pltpu.sync_copy(g_h.at[pl.ds(off, CV)], gi)
            pltpu.sync_copy(s_h.at[pl.ds(off, CV)], si)
            pltpu.sync_copy(a_h.at[pl.ds(off, CV)], al)
            pltpu.async_copy(t_h.at[gi], rb, sem)

        def process(gi, si, al, rb, sem):
            pltpu.make_async_copy(t_h.at[gi], rb, sem).wait()

            def mul(i, cc):
                a = al[i]
                for j in range(_L // 16):
                    sv = a.at[pats[j]].get(mode="promise_in_bounds")
                    sl = pl.ds(j * 16, 16)
                    rb[i, sl] = rb[i, sl] * sv
                return cc

            lax.fori_loop(0, CV, mul, 0)
            pltpu.sync_copy(rb, acc.at[si], add=True)

        load_issue(0, gi0, si0, al0, r0, sem0)

        def pairstep(t, carry):
            load_issue(2 * t + 1, gi1, si1, al1, r1, sem1)
            process(gi0, si0, al0, r0, sem0)
            nxt = lax.rem(2 * t + 2, nch)
            load_issue(nxt, gi0, si0, al0, r0, sem0)
            process(gi1, si1, al1, r1, sem1)
            return carry

        lax.fori_loop(0, nchH, pairstep, 0)
        pltpu.make_async_copy(t_h.at[gi0], r0, sem0).wait()
        plsc.subcore_barrier()

        def dump(r, carry):
            pltpu.sync_copy(acc.at[pl.ds(s * rpt + r * DCH, DCH)], dumpb)
            pltpu.sync_copy(dumpb, out_h.at[c, pl.ds(s * rpt + r * DCH, DCH)])
            return carry

        lax.fori_loop(0, ndc, dump, 0)

    return k(gidx, sidx, alpha, table,
             jnp.asarray(_PAT).reshape(_L // 16, 16))


def kernel(x, edge_index, hyperedge_attr, W_lin, b_lin, W_sheaf, b_sheaf,
           W1, b1, W2, b2, W_lin2, b_lin2):
    d, f = _D, _F
    df = d * f
    N = x.shape[0]
    M = hyperedge_attr.shape[0]
    E = edge_index.shape[1]

    row = edge_index[0].astype(jnp.int32)
    col = edge_index[1].astype(jnp.int32)

    W_top, W_bot = W_sheaf[:df], W_sheaf[df:]
    z10 = jnp.zeros((x.shape[1], 16 - d), jnp.float32)
    Wn = jnp.concatenate([W_lin, W_lin @ W_top, z10], axis=1)
    bn = jnp.concatenate([b_lin, b_lin @ W_top + b_sheaf,
                          jnp.zeros((16 - d,), jnp.float32)])[None]
    Wh = jnp.concatenate([W_lin, W_lin @ W_bot, z10], axis=1)
    bh = jnp.concatenate([b_lin, b_lin @ W_bot,
                          jnp.zeros((16 - d,), jnp.float32)])[None]
    xs_ext = _mm(x, Wn, bn)
    he_ext = _mm(hyperedge_attr, Wh, bh)
    srT = xs_ext[:, df:]
    scT = he_ext[:, df:]

    (row_p, col_p), _ = _pad_to_workers([row, col], 512)
    alpha16 = _alpha_sc(row_p, col_p, srT, scT, E)

    eye = jnp.eye(d, dtype=jnp.float32)
    pad10 = jnp.zeros((_L - df - d,), jnp.float32)
    b1t = jnp.concatenate([jnp.tile(b1, d), jnp.ones((d,), jnp.float32),
                           pad10])[None]
    b2t = jnp.concatenate([jnp.tile(b2, d), jnp.ones((d,), jnp.float32),
                           pad10])[None]
    W1x = jnp.zeros((70, _L), jnp.float32).at[:df, :df].set(jnp.kron(eye, W1))
    W2x = jnp.zeros((_L, _L), jnp.float32).at[:df, :df].set(jnp.kron(eye, W2))
    Wf = jnp.concatenate(
        [W_lin2, jnp.zeros((_L - df, W_lin2.shape[1]), jnp.float32)], axis=0)

    xp1T = _mm(xs_ext, W1x, b1t)
    p = _conv_sc(row_p, col_p, alpha16, xp1T, M)
    mT = _scale_m(p[0], p[1])
    q = _conv_sc(col_p, row_p, alpha16, mT, N)
    x1T = _scale_res(q[0], q[1], xp1T, elu=True)

    xp2T = _mm(x1T, W2x, b2t)
    p2 = _conv_sc(row_p, col_p, alpha16, xp2T, M)
    m2T = _scale_m(p2[0], p2[1])
    q2 = _conv_sc(col_p, row_p, alpha16, m2T, N)
    x2T = _scale_res(q2[0], q2[1], xp2T, elu=False)

    return _mm(x2T, Wf, b_lin2[None])
